# Initial kernel scaffold; baseline (speedup 1.0000x reference)
import jax
import jax.numpy as jnp
from jax.experimental import pallas as pl


def kernel(x, row_w, col_w):
    raise NotImplementedError("write your pallas kernel here")


# speedup vs baseline: 1.3188x; 1.3188x over previous
import functools

import jax
import jax.numpy as jnp
from jax import lax
from jax.experimental import pallas as pl
from jax.experimental.pallas import tpu as pltpu
from jax.experimental.pallas import tpu_sc as plsc

D_MODEL = 768
HALF = D_MODEL // 2
H = 256
W = 256
NW = 32
CPW = HALF // NW
REP_ROWS = 16
HALF_ROWS = 128


def _body(tab, out, row_v, rep_v, plane_v, sem):
    wid = lax.axis_index("s") * 2 + lax.axis_index("c")

    def t1_plane(k, carry):
        c = wid * CPW + k
        pltpu.sync_copy(tab.at[c], row_v)

        def fill_row(r, carry2):
            for j in range(W // 16):
                rep_v[r, pl.ds(j * 16, 16)] = row_v[pl.ds(j * 16, 16)]
            return carry2

        lax.fori_loop(0, REP_ROWS, fill_row, 0)

        def dma_block(b, carry2):
            pltpu.sync_copy(rep_v, out.at[c, pl.ds(b * REP_ROWS, REP_ROWS)])
            return carry2

        lax.fori_loop(0, H // REP_ROWS, dma_block, 0)
        return carry

    lax.fori_loop(0, CPW, t1_plane, 0)

    def t2_plane(k, carry):
        c = HALF + wid * CPW + k
        pltpu.sync_copy(tab.at[c], row_v)

        def do_half(h, carry2):
            def fill_chunk(cb, carry3):
                chunk = row_v[pl.ds(h * HALF_ROWS + cb * 16, 16)]
                for lane in range(16):
                    val = jnp.full((16,), chunk[lane])
                    r = cb * 16 + lane
                    for j in range(W // 16):
                        plane_v[r, pl.ds(j * 16, 16)] = val
                return carry3

            lax.fori_loop(0, HALF_ROWS // 16, fill_chunk, 0)
            pltpu.sync_copy(plane_v,
                            out.at[c, pl.ds(h * HALF_ROWS, HALF_ROWS)])
            return carry2

        lax.fori_loop(0, H // HALF_ROWS, do_half, 0)
        return carry

    lax.fori_loop(0, CPW, t2_plane, 0)


@functools.partial(jax.jit, static_argnums=())
def _pos_encode(tab):
    mesh = plsc.VectorSubcoreMesh(core_axis_name="c", subcore_axis_name="s")
    fn = functools.partial(
        pl.kernel,
        mesh=mesh,
        out_type=jax.ShapeDtypeStruct((D_MODEL, H, W), jnp.float32),
        scratch_types=[
            pltpu.VMEM((W,), jnp.float32),
            pltpu.VMEM((REP_ROWS, W), jnp.float32),
            pltpu.VMEM((HALF_ROWS, W), jnp.float32),
            pltpu.SemaphoreType.DMA,
        ],
    )(_body)
    return fn(tab)


def kernel(x, row_w, col_w):
    h = min(x.shape[-2], row_w.shape[0])
    w = min(x.shape[-1], col_w.shape[0])
    assert (h, w) == (H, W) and row_w.shape[1] == HALF
    tab = jnp.concatenate([col_w[:w].T, row_w[:h].T], axis=0)
    out = _pos_encode(tab)
    return out[None, ...]


# speedup vs baseline: 2.2430x; 1.7007x over previous
import functools

import jax
import jax.numpy as jnp
from jax import lax
from jax.experimental import pallas as pl
from jax.experimental.pallas import tpu as pltpu
from jax.experimental.pallas import tpu_sc as plsc

D_MODEL = 768
HALF = D_MODEL // 2
H = 256
W = 256
NW = 32
CPW = HALF // NW
REP_ROWS = 32
NREP = H // REP_ROWS
HALF_ROWS = 128
NJ = W // 16


def _body(tab, out, rows1_v, rows2_v, rep_v, hb0, hb1, sem_r, sem_h0, sem_h1):
    wid = lax.axis_index("s") * 2 + lax.axis_index("c")

    pltpu.sync_copy(tab.at[pl.ds(wid * CPW * W, CPW * W)], rows1_v)
    pltpu.sync_copy(tab.at[pl.ds((HALF + wid * CPW) * W, CPW * W)], rows2_v)

    def plane_pair(k, carry):
        c1 = wid * CPW + k
        c2 = HALF + wid * CPW + k

        chunks = [rows1_v[pl.ds(k * W + j * 16, 16)] for j in range(NJ)]

        def fill_rep(r, carry2):
            for j in range(NJ):
                rep_v[r, pl.ds(j * 16, 16)] = chunks[j]
            return carry2

        lax.fori_loop(0, REP_ROWS, fill_rep, 0)

        def fire_rep(d, carry2):
            pltpu.async_copy(
                rep_v, out.at[c1, pl.ds(d * REP_ROWS, REP_ROWS)], sem_r)
            return carry2

        lax.fori_loop(0, NREP, fire_rep, 0)

        for hbuf, sem, h in ((hb0, sem_h0, 0), (hb1, sem_h1, 1)):
            def fill_chunk(cb, carry2, hbuf=hbuf, h=h):
                chunk = rows2_v[pl.ds(k * W + h * HALF_ROWS + cb * 16, 16)]
                for lane in range(16):
                    val = jnp.full((16,), chunk[lane])
                    r = cb * 16 + lane
                    for j in range(NJ):
                        hbuf[r, pl.ds(j * 16, 16)] = val
                return carry2

            lax.fori_loop(0, HALF_ROWS // 16, fill_chunk, 0)
            pltpu.async_copy(
                hbuf, out.at[c2, pl.ds(h * HALF_ROWS, HALF_ROWS)], sem)

        def drain_rep(d, carry2):
            pltpu.make_async_copy(
                rep_v, out.at[c1, pl.ds(d * REP_ROWS, REP_ROWS)], sem_r
            ).wait()
            return carry2

        lax.fori_loop(0, NREP, drain_rep, 0)
        pltpu.make_async_copy(
            hb0, out.at[c2, pl.ds(0, HALF_ROWS)], sem_h0).wait()
        pltpu.make_async_copy(
            hb1, out.at[c2, pl.ds(HALF_ROWS, HALF_ROWS)], sem_h1).wait()
        return carry

    lax.fori_loop(0, CPW, plane_pair, 0)


@jax.jit
def _pos_encode(tab):
    mesh = plsc.VectorSubcoreMesh(core_axis_name="c", subcore_axis_name="s")
    fn = functools.partial(
        pl.kernel,
        mesh=mesh,
        out_type=jax.ShapeDtypeStruct((D_MODEL, H, W), jnp.float32),
        scratch_types=[
            pltpu.VMEM((CPW * W,), jnp.float32),
            pltpu.VMEM((CPW * W,), jnp.float32),
            pltpu.VMEM((REP_ROWS, W), jnp.float32),
            pltpu.VMEM((HALF_ROWS, W), jnp.float32),
            pltpu.VMEM((HALF_ROWS, W), jnp.float32),
            pltpu.SemaphoreType.DMA,
            pltpu.SemaphoreType.DMA,
            pltpu.SemaphoreType.DMA,
        ],
    )(_body)
    return fn(tab)


def kernel(x, row_w, col_w):
    h = min(x.shape[-2], row_w.shape[0])
    w = min(x.shape[-1], col_w.shape[0])
    assert (h, w) == (H, W) and row_w.shape[1] == HALF
    tab = jnp.concatenate([col_w[:w].T, row_w[:h].T], axis=0).reshape(-1)
    out = _pos_encode(tab)
    return out[None, ...]


# speedup vs baseline: 2.3073x; 1.0286x over previous
import functools

import jax
import jax.numpy as jnp
from jax import lax
from jax.experimental import pallas as pl
from jax.experimental.pallas import tpu as pltpu
from jax.experimental.pallas import tpu_sc as plsc

D_MODEL = 768
HALF = D_MODEL // 2
H = 256
W = 256
NW = 32
CPW = HALF // NW
REP_ROWS = 32
NREP = H // REP_ROWS
STRIPE_W = 128
NJ = W // 16


def _body(tab, out, rows1_v, rows2_v, rep_v, stripe_v, sem_r, sem_h0, sem_h1):
    wid = lax.axis_index("s") * 2 + lax.axis_index("c")

    pltpu.sync_copy(tab.at[pl.ds(wid * CPW * W, CPW * W)], rows1_v)
    pltpu.sync_copy(tab.at[pl.ds((HALF + wid * CPW) * W, CPW * W)], rows2_v)

    def plane_pair(k, carry):
        c1 = wid * CPW + k
        c2 = HALF + wid * CPW + k

        chunks = [rows1_v[pl.ds(k * W + j * 16, 16)] for j in range(NJ)]

        def fill_rep(r, carry2):
            for j in range(NJ):
                rep_v[r, pl.ds(j * 16, 16)] = chunks[j]
            return carry2

        lax.fori_loop(0, REP_ROWS, fill_rep, 0)

        def fire_rep(d, carry2):
            pltpu.async_copy(
                rep_v, out.at[c1, pl.ds(d * REP_ROWS, REP_ROWS)], sem_r)
            return carry2

        lax.fori_loop(0, NREP, fire_rep, 0)

        def fill_chunk(cb, carry2):
            chunk = rows2_v[pl.ds(k * W + cb * 16, 16)]
            for lane in range(16):
                val = jnp.full((16,), chunk[lane])
                r = cb * 16 + lane
                for j in range(STRIPE_W // 16):
                    stripe_v[r, pl.ds(j * 16, 16)] = val
            return carry2

        lax.fori_loop(0, H // 16, fill_chunk, 0)
        pltpu.async_copy(stripe_v, out.at[c2, :, pl.ds(0, STRIPE_W)], sem_h0)
        pltpu.async_copy(
            stripe_v, out.at[c2, :, pl.ds(STRIPE_W, STRIPE_W)], sem_h1)

        def drain_rep(d, carry2):
            pltpu.make_async_copy(
                rep_v, out.at[c1, pl.ds(d * REP_ROWS, REP_ROWS)], sem_r
            ).wait()
            return carry2

        lax.fori_loop(0, NREP, drain_rep, 0)
        pltpu.make_async_copy(
            stripe_v, out.at[c2, :, pl.ds(0, STRIPE_W)], sem_h0).wait()
        pltpu.make_async_copy(
            stripe_v, out.at[c2, :, pl.ds(STRIPE_W, STRIPE_W)], sem_h1).wait()
        return carry

    lax.fori_loop(0, CPW, plane_pair, 0)


@jax.jit
def _pos_encode(tab):
    mesh = plsc.VectorSubcoreMesh(core_axis_name="c", subcore_axis_name="s")
    fn = functools.partial(
        pl.kernel,
        mesh=mesh,
        out_type=jax.ShapeDtypeStruct((D_MODEL, H, W), jnp.float32),
        scratch_types=[
            pltpu.VMEM((CPW * W,), jnp.float32),
            pltpu.VMEM((CPW * W,), jnp.float32),
            pltpu.VMEM((REP_ROWS, W), jnp.float32),
            pltpu.VMEM((H, STRIPE_W), jnp.float32),
            pltpu.SemaphoreType.DMA,
            pltpu.SemaphoreType.DMA,
            pltpu.SemaphoreType.DMA,
        ],
    )(_body)
    return fn(tab)


def kernel(x, row_w, col_w):
    h = min(x.shape[-2], row_w.shape[0])
    w = min(x.shape[-1], col_w.shape[0])
    assert (h, w) == (H, W) and row_w.shape[1] == HALF
    tab = jnp.concatenate([col_w[:w].T, row_w[:h].T], axis=0).reshape(-1)
    out = _pos_encode(tab)
    return out[None, ...]
